# baseline (speedup 1.0000x reference)
import functools

import jax
import jax.numpy as jnp
import numpy as np
from jax import lax
from jax.experimental import pallas as pl
from jax.experimental.pallas import tpu as pltpu
from jax.experimental.pallas import tpu_sc as plsc

_TOP_C = 1024
_MIN_I32 = np.int32(-(2 ** 31))


def _sortable_keys(x):
    b = lax.bitcast_convert_type(x, jnp.int32)
    return jnp.where(b < 0, jnp.bitwise_xor(jnp.bitwise_not(b), _MIN_I32), b)


def _inclusive_cumsum0(x):
    n = x.shape[0]
    d = 1
    while d < n:
        pad = jnp.zeros((d,) + x.shape[1:], x.dtype)
        x = x + jnp.concatenate([pad, x[:-d]], axis=0)
        d *= 2
    return x


def _mm_body(x_ref, w_ref, o_ref):
    r = jnp.dot(x_ref[...], w_ref[...], preferred_element_type=jnp.float32)
    o_ref[...] = jnp.where(r == 0.0, jnp.float32(0.0), r)


def _mm_body_t(x_ref, w_ref, o_ref, ot_ref):
    r = jnp.dot(x_ref[...], w_ref[...], preferred_element_type=jnp.float32)
    r = jnp.where(r == 0.0, jnp.float32(0.0), r)
    o_ref[...] = r
    ot_ref[...] = r.T


def _thresh_body(*refs, C, E, F):
    lg_refs = refs[:F]
    thr_ref, keq_ref, hi_ref, lo_ref = refs[F:F + 4]
    keys = jnp.concatenate([_sortable_keys(r[...]) for r in lg_refs], axis=1)
    hi_ref[...] = lax.shift_right_arithmetic(keys, 16).astype(jnp.int16)
    lo_ref[...] = jnp.bitwise_xor(keys.astype(jnp.int16),
                                  np.int16(-(2 ** 15)))
    n2 = hi_ref.shape[0]
    ones8 = jnp.ones((8, n2), jnp.bfloat16)
    cf32 = jnp.float32(C)

    def count(ref, cand_u, strict=False):
        cs = jnp.bitwise_xor(cand_u, jnp.int32(2 ** 15)).astype(jnp.int16)
        cf = jnp.concatenate([cs] * F, axis=1)
        a = ref[...]
        m = (a > cf) if strict else (a >= cf)
        mb = jnp.where(m, jnp.bfloat16(1.0), jnp.bfloat16(0.0))
        c8 = lax.dot_general(ones8, mb, (((1,), (0,)), ((), ())),
                             preferred_element_type=jnp.float32)
        c = c8[0:1]
        return sum(c[:, i * E:(i + 1) * E] for i in range(F))

    def search(ref, need):
        def body(i, t_u):
            bitval = lax.shift_left(jnp.int32(1), jnp.int32(15) - i)
            cand_u = jnp.bitwise_or(t_u, bitval)
            return jnp.where(count(ref, cand_u) >= need, cand_u, t_u)
        return lax.fori_loop(0, 16, body, jnp.zeros((1, E), jnp.int32))

    t_hi_u = search(hi_ref, cf32)
    g_hi = count(hi_ref, t_hi_u, strict=True)
    c2 = cf32 - g_hi

    t_hi_s = jnp.bitwise_xor(t_hi_u, jnp.int32(2 ** 15)).astype(jnp.int16)
    t_hi_b = jnp.concatenate([t_hi_s] * F, axis=1)
    lo_ref[...] = jnp.where(hi_ref[...] == t_hi_b, lo_ref[...],
                            np.int16(-(2 ** 15)))

    t_lo_u = search(lo_ref, c2)
    g2 = count(lo_ref, t_lo_u, strict=True)

    t_u = jnp.bitwise_or(lax.shift_left(t_hi_u, 16), t_lo_u)
    thr_ref[...] = jnp.bitwise_xor(t_u, _MIN_I32)
    keq_ref[...] = (c2 - g2).astype(jnp.int32)


_SC_STAGES = 5


def _sc_thresh(e, n, C):
    info = plsc.get_sparse_core_info()
    nw = info.num_cores * info.num_subcores
    epw = e // nw
    nv = n // 16
    mesh = plsc.VectorSubcoreMesh(core_axis_name="c", subcore_axis_name="s")

    @functools.partial(
        pl.kernel, mesh=mesh,
        out_type=jax.ShapeDtypeStruct((nw, 16), jnp.int32),
        compiler_params=pltpu.CompilerParams(
            needs_layout_passes=False, use_tc_tiling_on_sc=True),
        scratch_types=[
            pltpu.VMEM((n,), jnp.float32),
            pltpu.VMEM((n,), jnp.int32),
            pltpu.VMEM((256,), jnp.int32),
            pltpu.VMEM((16,), jnp.int32),
        ],
    )
    def k(lt_hbm, out_hbm, stage_v, keys_v, hist_v, res_v):
        wid = lax.axis_index("s") * info.num_cores + lax.axis_index("c")
        ones16 = jnp.ones((16,), jnp.int32)
        zeros16 = jnp.zeros((16,), jnp.int32)
        idx16 = lax.iota(jnp.int32, 16)
        res_v[...] = zeros16

        for ei in range(epw if _SC_STAGES >= 1 else 0):
            erow = wid * epw + ei
            pltpu.sync_copy(lt_hbm.at[erow], stage_v)

            @plsc.parallel_loop(0, n, step=16, unroll=16)
            def _(i):
                v = stage_v[pl.ds(i, 16)]
                b = lax.bitcast_convert_type(v, jnp.int32)
                keys_v[pl.ds(i, 16)] = jnp.where(
                    b < 0, jnp.bitwise_not(b),
                    jnp.bitwise_xor(b, _MIN_I32))

            prefix = jnp.int32(0)
            need = jnp.int32(C)
            for pi, shift in enumerate((24, 16, 8, 0) if _SC_STAGES >= 4
                                       else ()):
                for j in range(16):
                    hist_v[pl.ds(j * 16, 16)] = zeros16

                pfx = prefix

                @plsc.parallel_loop(0, n, step=16, unroll=16)
                def _(i, shift=shift, with_prefix=(pi > 0), pfx=pfx):
                    ku = keys_v[pl.ds(i, 16)]
                    dig = jnp.bitwise_and(
                        lax.shift_right_logical(ku, shift), 255)
                    if with_prefix:
                        m = lax.shift_right_logical(ku, shift + 8) == pfx
                        plsc.addupdate_scatter(hist_v, [dig], ones16, mask=m)
                    else:
                        plsc.addupdate_scatter(hist_v, [dig], ones16)

                cum = jnp.int32(0)
                bin_acc = jnp.int32(0)
                above_acc = jnp.int32(0)
                for j in range(16 if _SC_STAGES >= 5 else 0):
                    base = 256 - 16 * (j + 1)
                    v = hist_v[pl.ds(base, 16)]
                    rv = lax.rev(v, (0,))
                    cs = plsc.cumsum(rv)
                    g = cs + cum
                    sfx = g - rv
                    hit = jnp.logical_and(sfx < need, g >= need)
                    bins = jnp.full((16,), base + 15, jnp.int32) - idx16
                    bin_acc = bin_acc + jnp.sum(jnp.where(hit, bins, 0))
                    above_acc = above_acc + jnp.sum(jnp.where(hit, sfx, 0))
                    cum = cum + jnp.sum(v)
                prefix = jnp.bitwise_or(lax.shift_left(prefix, 8), bin_acc)
                need = need - above_acc

            if _SC_STAGES >= 3:
                t_s = jnp.bitwise_xor(prefix, _MIN_I32)
                r = res_v[...]
                r = jnp.where(idx16 == 2 * ei, t_s, r)
                r = jnp.where(idx16 == 2 * ei + 1, need, r)
                res_v[...] = r

        pltpu.sync_copy(res_v, out_hbm.at[wid])

    return k, nw, epw


def _final_body(lg_ref, thr_ref, keq_ref, probs_ref, rout_ref, aux_ref,
                runq_ref, imp_ref, load_ref, *, E, nb):
    i = pl.program_id(0)

    @pl.when(i == 0)
    def _():
        runq_ref[...] = jnp.zeros_like(runq_ref)
        imp_ref[...] = jnp.zeros_like(imp_ref)
        load_ref[...] = jnp.zeros_like(load_ref)

    l = lg_ref[...]
    thr = thr_ref[...]
    thr_bits = jnp.where(thr >= 0, thr,
                         jnp.bitwise_not(jnp.bitwise_xor(thr, _MIN_I32)))
    thr_f = lax.bitcast_convert_type(thr_bits, jnp.float32)
    keq = keq_ref[...]
    gt = l > thr_f
    eq = l == thr_f
    eq_i = eq.astype(jnp.int32)
    eq_col = jnp.sum(eq_i, axis=0, keepdims=True)
    s_eq = jnp.sum(eq_col)

    @pl.when(s_eq == 0)
    def _():
        rout_ref[...] = jnp.where(gt, jnp.float32(1.0),
                                  jnp.float32(0.0))[None]

    @pl.when(s_eq > 0)
    def _():
        prefix = _inclusive_cumsum0(eq_i) - eq_i + runq_ref[...]
        take_eq = jnp.logical_and(eq, prefix < keq)
        rout_ref[...] = jnp.where(jnp.logical_or(gt, take_eq),
                                  jnp.float32(1.0), jnp.float32(0.0))[None]

    runq_ref[...] = runq_ref[...] + eq_col
    routing = rout_ref[...][0]
    ones_e = jnp.ones((E, E), jnp.bfloat16)

    def rowsum(x):
        x_hi = x.astype(jnp.bfloat16)
        x_lo = (x - x_hi.astype(jnp.float32)).astype(jnp.bfloat16)
        d = lax.dot_general(x_hi, ones_e, (((1,), (0,)), ((), ())),
                            preferred_element_type=jnp.float32)
        d_lo = lax.dot_general(x_lo, ones_e, (((1,), (0,)), ((), ())),
                               preferred_element_type=jnp.float32)
        return d + d_lo

    p = jnp.exp(l)
    zb = rowsum(p)
    probs = p / zb
    masked = probs * routing
    denom = rowsum(masked) + jnp.float32(1e-6)
    outp = masked / denom
    probs_ref[...] = outp[None]
    imp_ref[...] = imp_ref[...] + jnp.sum(outp, axis=0, keepdims=True)
    load_ref[...] = load_ref[...] + jnp.sum(routing, axis=0, keepdims=True)

    @pl.when(i == nb - 1)
    def _():
        def cv2(x):
            mean = jnp.sum(x) / E
            var = jnp.sum((x - mean) ** 2) / (E - 1)
            return var / (mean + jnp.float32(1e-6)) ** 2

        aux_ref[...] = jnp.reshape(cv2(imp_ref[...]) + cv2(load_ref[...]),
                                   (1, 1))


def _router(flat, w, C, bs, blk_mm=4096, blk_fin=1024, sc=False):
    n, h = flat.shape
    e = w.shape[1]
    nb_seq = bs[1] // blk_fin
    f = max(1, 128 // e)
    assert n % blk_mm == 0 and n % blk_fin == 0 and n % f == 0

    if sc:
        logits, logits_t = pl.pallas_call(
            _mm_body_t,
            grid=(n // blk_mm,),
            in_specs=[pl.BlockSpec((blk_mm, h), lambda i: (i, 0)),
                      pl.BlockSpec((h, e), lambda i: (0, 0))],
            out_specs=[pl.BlockSpec((blk_mm, e), lambda i: (i, 0)),
                       pl.BlockSpec((e, blk_mm), lambda i: (0, i))],
            out_shape=[jax.ShapeDtypeStruct((n, e), jnp.float32),
                       jax.ShapeDtypeStruct((e, n), jnp.float32)],
            compiler_params=pltpu.CompilerParams(
                dimension_semantics=("parallel",)),
        )(flat, w)
        sck, nw, epw = _sc_thresh(e, n, C)
        res = sck(logits_t)
        pairs = res[:, :2 * epw].reshape(e, 2)
        thr = pairs[:, 0].reshape(1, e)
        keq = pairs[:, 1].reshape(1, e)
    else:
        logits = pl.pallas_call(
            _mm_body,
            grid=(n // blk_mm,),
            in_specs=[pl.BlockSpec((blk_mm, h), lambda i: (i, 0)),
                      pl.BlockSpec((h, e), lambda i: (0, 0))],
            out_specs=pl.BlockSpec((blk_mm, e), lambda i: (i, 0)),
            out_shape=jax.ShapeDtypeStruct((n, e), jnp.float32),
            compiler_params=pltpu.CompilerParams(
                dimension_semantics=("parallel",)),
        )(flat, w)

        def _view_map(view):
            return lambda i, view=view: (view, 0)

        thr, keq = pl.pallas_call(
            functools.partial(_thresh_body, C=C, E=e, F=f),
            grid=(1,),
            in_specs=[pl.BlockSpec((n // f, e), _view_map(v))
                      for v in range(f)],
            out_specs=[pl.BlockSpec((1, e), lambda i: (0, 0)),
                       pl.BlockSpec((1, e), lambda i: (0, 0))],
            out_shape=[jax.ShapeDtypeStruct((1, e), jnp.int32),
                       jax.ShapeDtypeStruct((1, e), jnp.int32)],
            scratch_shapes=[pltpu.VMEM((n // f, e * f), jnp.int16),
                            pltpu.VMEM((n // f, e * f), jnp.int16)],
        )(*([logits] * f))

    nb = n // blk_fin
    probs, routing, aux = pl.pallas_call(
        functools.partial(_final_body, E=e, nb=nb),
        grid=(nb,),
        in_specs=[pl.BlockSpec((blk_fin, e), lambda i: (i, 0)),
                  pl.BlockSpec((1, e), lambda i: (0, 0)),
                  pl.BlockSpec((1, e), lambda i: (0, 0))],
        out_specs=[pl.BlockSpec((1, blk_fin, e),
                                lambda i: (i // nb_seq, i % nb_seq, 0)),
                   pl.BlockSpec((1, blk_fin, e),
                                lambda i: (i // nb_seq, i % nb_seq, 0)),
                   pl.BlockSpec((1, 1), lambda i: (0, 0))],
        out_shape=[jax.ShapeDtypeStruct((bs[0], bs[1], e), jnp.float32),
                   jax.ShapeDtypeStruct((bs[0], bs[1], e), jnp.float32),
                   jax.ShapeDtypeStruct((1, 1), jnp.float32)],
        scratch_shapes=[pltpu.VMEM((1, e), jnp.int32),
                        pltpu.VMEM((1, e), jnp.float32),
                        pltpu.VMEM((1, e), jnp.float32)],
        compiler_params=pltpu.CompilerParams(
            dimension_semantics=("arbitrary",)),
    )(logits, thr, keq)

    return probs, routing, aux


def kernel(hidden_states, W):
    b, s, h = hidden_states.shape
    e = W.shape[1]
    flat = hidden_states.reshape(b * s, h)
    probs, routing, aux = _router(flat, W, _TOP_C, (b, s), sc=True)
    return (probs, routing, aux[0, 0])


# speedup vs baseline: 1.0286x; 1.0286x over previous
import functools

import jax
import jax.numpy as jnp
import numpy as np
from jax import lax
from jax.experimental import pallas as pl
from jax.experimental.pallas import tpu as pltpu
from jax.experimental.pallas import tpu_sc as plsc

_TOP_C = 1024
_MIN_I32 = np.int32(-(2 ** 31))


def _inclusive_cumsum0(x):
    n = x.shape[0]
    d = 1
    while d < n:
        pad = jnp.zeros((d,) + x.shape[1:], x.dtype)
        x = x + jnp.concatenate([pad, x[:-d]], axis=0)
        d *= 2
    return x


def _mm_body_t(x_ref, w_ref, o_ref, ot_ref):
    r = jnp.dot(x_ref[...], w_ref[...], preferred_element_type=jnp.float32)
    r = jnp.where(r == 0.0, jnp.float32(0.0), r)
    o_ref[...] = r
    ot_ref[...] = r.T


def _sc_thresh(e, n, C):
    info = plsc.get_sparse_core_info()
    nw = info.num_cores * info.num_subcores
    epw = e // nw
    nv = n // 16
    mesh = plsc.VectorSubcoreMesh(core_axis_name="c", subcore_axis_name="s")

    @functools.partial(
        pl.kernel, mesh=mesh,
        out_type=jax.ShapeDtypeStruct((nw, 16), jnp.int32),
        compiler_params=pltpu.CompilerParams(needs_layout_passes=False),
        scratch_types=[
            pltpu.VMEM((n,), jnp.float32),
            pltpu.VMEM((n,), jnp.int32),
            pltpu.VMEM((256,), jnp.int32),
            pltpu.VMEM((16,), jnp.int32),
        ],
    )
    def k(lt_hbm, out_hbm, stage_v, keys_v, hist_v, res_v):
        wid = lax.axis_index("s") * info.num_cores + lax.axis_index("c")
        ones16 = jnp.ones((16,), jnp.int32)
        zeros16 = jnp.zeros((16,), jnp.int32)
        idx16 = lax.iota(jnp.int32, 16)
        res_v[...] = zeros16

        for ei in range(epw):
            erow = wid * epw + ei
            pltpu.sync_copy(lt_hbm.at[erow], stage_v)

            @plsc.parallel_loop(0, n, step=16, unroll=16)
            def _(i):
                v = stage_v[pl.ds(i, 16)]
                b = lax.bitcast_convert_type(v, jnp.int32)
                keys_v[pl.ds(i, 16)] = jnp.where(
                    b < 0, jnp.bitwise_not(b),
                    jnp.bitwise_xor(b, _MIN_I32))

            prefix = jnp.int32(0)
            need = jnp.int32(C)
            for pi, shift in enumerate((24, 16, 8, 0)):
                for j in range(16):
                    hist_v[pl.ds(j * 16, 16)] = zeros16

                pfx = prefix

                @plsc.parallel_loop(0, n, step=16, unroll=16)
                def _(i, shift=shift, with_prefix=(pi > 0), pfx=pfx):
                    ku = keys_v[pl.ds(i, 16)]
                    dig = jnp.bitwise_and(
                        lax.shift_right_logical(ku, shift), 255)
                    if with_prefix:
                        m = lax.shift_right_logical(ku, shift + 8) == pfx
                        plsc.addupdate_scatter(hist_v, [dig], ones16, mask=m)
                    else:
                        plsc.addupdate_scatter(hist_v, [dig], ones16)

                cum = jnp.int32(0)
                bin_acc = jnp.int32(0)
                above_acc = jnp.int32(0)
                for j in range(16):
                    base = 256 - 16 * (j + 1)
                    v = hist_v[pl.ds(base, 16)]
                    rv = lax.rev(v, (0,))
                    cs = plsc.cumsum(rv)
                    g = cs + cum
                    sfx = g - rv
                    hit = jnp.logical_and(sfx < need, g >= need)
                    bins = jnp.full((16,), base + 15, jnp.int32) - idx16
                    bin_acc = bin_acc + jnp.sum(jnp.where(hit, bins, 0))
                    above_acc = above_acc + jnp.sum(jnp.where(hit, sfx, 0))
                    cum = cum + jnp.sum(v)
                prefix = jnp.bitwise_or(lax.shift_left(prefix, 8), bin_acc)
                need = need - above_acc

            t_s = jnp.bitwise_xor(prefix, _MIN_I32)
            r = res_v[...]
            r = jnp.where(idx16 == 2 * ei, t_s, r)
            r = jnp.where(idx16 == 2 * ei + 1, need, r)
            res_v[...] = r

        pltpu.sync_copy(res_v, out_hbm.at[wid])

    return k, nw, epw


def _final_body(lg_ref, thr_ref, keq_ref, probs_ref, rout_ref, aux_ref,
                runq_ref, imp_ref, load_ref, *, E, nb):
    i = pl.program_id(0)

    @pl.when(i == 0)
    def _():
        runq_ref[...] = jnp.zeros_like(runq_ref)
        imp_ref[...] = jnp.zeros_like(imp_ref)
        load_ref[...] = jnp.zeros_like(load_ref)

    l = lg_ref[...]
    thr = thr_ref[...]
    thr_bits = jnp.where(thr >= 0, thr,
                         jnp.bitwise_not(jnp.bitwise_xor(thr, _MIN_I32)))
    thr_f = lax.bitcast_convert_type(thr_bits, jnp.float32)
    keq = keq_ref[...]
    gt = l > thr_f
    eq = l == thr_f
    eq_i = eq.astype(jnp.int32)
    eq_col = jnp.sum(eq_i, axis=0, keepdims=True)
    s_eq = jnp.sum(eq_col)

    @pl.when(s_eq == 0)
    def _():
        rout_ref[...] = jnp.where(gt, jnp.float32(1.0), jnp.float32(0.0))

    @pl.when(s_eq > 0)
    def _():
        prefix = _inclusive_cumsum0(eq_i) - eq_i + runq_ref[...]
        take_eq = jnp.logical_and(eq, prefix < keq)
        rout_ref[...] = jnp.where(jnp.logical_or(gt, take_eq),
                                  jnp.float32(1.0), jnp.float32(0.0))

    runq_ref[...] = runq_ref[...] + eq_col
    routing = rout_ref[...]
    ones_e = jnp.ones((E, E), jnp.bfloat16)

    def rowsum(x):
        x_hi = x.astype(jnp.bfloat16)
        x_lo = (x - x_hi.astype(jnp.float32)).astype(jnp.bfloat16)
        d = lax.dot_general(x_hi, ones_e, (((1,), (0,)), ((), ())),
                            preferred_element_type=jnp.float32)
        d_lo = lax.dot_general(x_lo, ones_e, (((1,), (0,)), ((), ())),
                               preferred_element_type=jnp.float32)
        return d + d_lo

    p = jnp.exp(l)
    zb = rowsum(p)
    probs = p / zb
    masked = probs * routing
    denom = rowsum(masked) + jnp.float32(1e-6)
    outp = masked / denom
    probs_ref[...] = outp
    imp_ref[...] = imp_ref[...] + jnp.sum(outp, axis=0, keepdims=True)
    load_ref[...] = load_ref[...] + jnp.sum(routing, axis=0, keepdims=True)

    @pl.when(i == nb - 1)
    def _():
        def cv2(x):
            mean = jnp.sum(x) / E
            var = jnp.sum((x - mean) ** 2) / (E - 1)
            return var / (mean + jnp.float32(1e-6)) ** 2

        aux_ref[...] = jnp.reshape(cv2(imp_ref[...]) + cv2(load_ref[...]),
                                   (1, 1))


def _router(flat, w, C, blk_mm=4096, blk_fin=1024):
    n, h = flat.shape
    e = w.shape[1]
    assert n % blk_mm == 0 and n % blk_fin == 0

    logits, logits_t = pl.pallas_call(
        _mm_body_t,
        grid=(n // blk_mm,),
        in_specs=[pl.BlockSpec((blk_mm, h), lambda i: (i, 0)),
                  pl.BlockSpec((h, e), lambda i: (0, 0))],
        out_specs=[pl.BlockSpec((blk_mm, e), lambda i: (i, 0)),
                   pl.BlockSpec((e, blk_mm), lambda i: (0, i))],
        out_shape=[jax.ShapeDtypeStruct((n, e), jnp.float32),
                   jax.ShapeDtypeStruct((e, n), jnp.float32)],
        compiler_params=pltpu.CompilerParams(
            dimension_semantics=("parallel",)),
    )(flat, w)
    sck, nw, epw = _sc_thresh(e, n, C)
    res = sck(logits_t)
    pairs = res[:, :2 * epw].reshape(e, 2)
    thr = pairs[:, 0].reshape(1, e)
    keq = pairs[:, 1].reshape(1, e)

    nb = n // blk_fin
    probs, routing, aux = pl.pallas_call(
        functools.partial(_final_body, E=e, nb=nb),
        grid=(nb,),
        in_specs=[pl.BlockSpec((blk_fin, e), lambda i: (i, 0)),
                  pl.BlockSpec((1, e), lambda i: (0, 0)),
                  pl.BlockSpec((1, e), lambda i: (0, 0))],
        out_specs=[pl.BlockSpec((blk_fin, e), lambda i: (i, 0)),
                   pl.BlockSpec((blk_fin, e), lambda i: (i, 0)),
                   pl.BlockSpec((1, 1), lambda i: (0, 0))],
        out_shape=[jax.ShapeDtypeStruct((n, e), jnp.float32),
                   jax.ShapeDtypeStruct((n, e), jnp.float32),
                   jax.ShapeDtypeStruct((1, 1), jnp.float32)],
        scratch_shapes=[pltpu.VMEM((1, e), jnp.int32),
                        pltpu.VMEM((1, e), jnp.float32),
                        pltpu.VMEM((1, e), jnp.float32)],
        compiler_params=pltpu.CompilerParams(
            dimension_semantics=("arbitrary",)),
    )(logits, thr, keq)

    return probs, routing, aux


def kernel(hidden_states, W):
    b, s, h = hidden_states.shape
    e = W.shape[1]
    flat = hidden_states.reshape(b * s, h)
    probs, routing, aux = _router(flat, W, _TOP_C)
    return (probs.reshape(b, s, e), routing.reshape(b, s, e), aux[0, 0])


# speedup vs baseline: 1.0291x; 1.0005x over previous
import functools

import jax
import jax.numpy as jnp
import numpy as np
from jax import lax
from jax.experimental import pallas as pl
from jax.experimental.pallas import tpu as pltpu
from jax.experimental.pallas import tpu_sc as plsc

_TOP_C = 1024
_MIN_I32 = np.int32(-(2 ** 31))


def _inclusive_cumsum0(x):
    n = x.shape[0]
    d = 1
    while d < n:
        pad = jnp.zeros((d,) + x.shape[1:], x.dtype)
        x = x + jnp.concatenate([pad, x[:-d]], axis=0)
        d *= 2
    return x


def _mm_body_t(x_ref, w_ref, o_ref, ot_ref):
    r = jnp.dot(x_ref[...], w_ref[...], preferred_element_type=jnp.float32)
    r = jnp.where(r == 0.0, jnp.float32(0.0), r)
    o_ref[...] = r
    ot_ref[...] = r.T


def _sc_thresh(e, n, C):
    info = plsc.get_sparse_core_info()
    nw = info.num_cores * info.num_subcores
    epw = e // nw
    mesh = plsc.VectorSubcoreMesh(core_axis_name="c", subcore_axis_name="s")

    @functools.partial(
        pl.kernel, mesh=mesh,
        out_type=jax.ShapeDtypeStruct((nw, 16), jnp.int32),
        compiler_params=pltpu.CompilerParams(needs_layout_passes=False),
        scratch_types=[
            pltpu.VMEM((n,), jnp.float32),
            pltpu.VMEM((n,), jnp.int32),
            pltpu.VMEM((256,), jnp.int32),
            pltpu.VMEM((16,), jnp.int32),
        ],
    )
    def k(lt_hbm, out_hbm, stage_v, keys_v, hist_v, res_v):
        wid = lax.axis_index("s") * info.num_cores + lax.axis_index("c")
        ones16 = jnp.ones((16,), jnp.int32)
        zeros16 = jnp.zeros((16,), jnp.int32)
        idx16 = lax.iota(jnp.int32, 16)
        res_v[...] = zeros16

        for ei in range(epw):
            erow = wid * epw + ei
            pltpu.sync_copy(lt_hbm.at[erow], stage_v)

            @plsc.parallel_loop(0, n, step=16, unroll=16)
            def _(i):
                v = stage_v[pl.ds(i, 16)]
                b = lax.bitcast_convert_type(v, jnp.int32)
                keys_v[pl.ds(i, 16)] = jnp.where(
                    b < 0, jnp.bitwise_not(b),
                    jnp.bitwise_xor(b, _MIN_I32))

            prefix = jnp.int32(0)
            need = jnp.int32(C)
            for pi, shift in enumerate((24, 16, 8, 0)):
                for j in range(16):
                    hist_v[pl.ds(j * 16, 16)] = zeros16

                pfx = prefix

                @plsc.parallel_loop(0, n, step=16, unroll=16)
                def _(i, shift=shift, with_prefix=(pi > 0), pfx=pfx):
                    ku = keys_v[pl.ds(i, 16)]
                    dig = jnp.bitwise_and(
                        lax.shift_right_logical(ku, shift), 255)
                    if with_prefix:
                        m = lax.shift_right_logical(ku, shift + 8) == pfx
                        plsc.addupdate_scatter(hist_v, [dig], ones16, mask=m)
                    else:
                        plsc.addupdate_scatter(hist_v, [dig], ones16)

                cum = jnp.int32(0)
                bin_acc = jnp.int32(0)
                above_acc = jnp.int32(0)
                for j in range(16):
                    base = 256 - 16 * (j + 1)
                    v = hist_v[pl.ds(base, 16)]
                    rv = lax.rev(v, (0,))
                    cs = plsc.cumsum(rv)
                    g = cs + cum
                    sfx = g - rv
                    hit = jnp.logical_and(sfx < need, g >= need)
                    bins = jnp.full((16,), base + 15, jnp.int32) - idx16
                    bin_acc = bin_acc + jnp.sum(jnp.where(hit, bins, 0))
                    above_acc = above_acc + jnp.sum(jnp.where(hit, sfx, 0))
                    cum = cum + jnp.sum(v)
                prefix = jnp.bitwise_or(lax.shift_left(prefix, 8), bin_acc)
                need = need - above_acc

            t_s = jnp.bitwise_xor(prefix, _MIN_I32)
            r = res_v[...]
            r = jnp.where(idx16 == 2 * ei, t_s, r)
            r = jnp.where(idx16 == 2 * ei + 1, need, r)
            res_v[...] = r

        pltpu.sync_copy(res_v, out_hbm.at[wid])

    return k, nw, epw


def _final_body(lg_ref, thr_ref, keq_ref, probs_ref, rout_ref, aux_ref,
                runq_ref, imp_ref, load_ref, *, E, nb):
    i = pl.program_id(0)

    @pl.when(i == 0)
    def _():
        runq_ref[...] = jnp.zeros_like(runq_ref)
        imp_ref[...] = jnp.zeros_like(imp_ref)
        load_ref[...] = jnp.zeros_like(load_ref)

    l = lg_ref[...]
    thr = thr_ref[...]
    thr_bits = jnp.where(thr >= 0, thr,
                         jnp.bitwise_not(jnp.bitwise_xor(thr, _MIN_I32)))
    thr_f = lax.bitcast_convert_type(thr_bits, jnp.float32)
    keq = keq_ref[...]
    gt = l > thr_f
    eq = l == thr_f
    eq_i = eq.astype(jnp.int32)
    eq_col = jnp.sum(eq_i, axis=0, keepdims=True)
    s_eq = jnp.sum(eq_col)

    @pl.when(s_eq == 0)
    def _():
        rout_ref[...] = jnp.where(gt, jnp.float32(1.0), jnp.float32(0.0))

    @pl.when(s_eq > 0)
    def _():
        prefix = _inclusive_cumsum0(eq_i) - eq_i + runq_ref[...]
        take_eq = jnp.logical_and(eq, prefix < keq)
        rout_ref[...] = jnp.where(jnp.logical_or(gt, take_eq),
                                  jnp.float32(1.0), jnp.float32(0.0))

    runq_ref[...] = runq_ref[...] + eq_col
    routing = rout_ref[...]
    ones_e = jnp.ones((E, E), jnp.bfloat16)

    def rowsum(x):
        x_hi = x.astype(jnp.bfloat16)
        x_lo = (x - x_hi.astype(jnp.float32)).astype(jnp.bfloat16)
        d = lax.dot_general(x_hi, ones_e, (((1,), (0,)), ((), ())),
                            preferred_element_type=jnp.float32)
        d_lo = lax.dot_general(x_lo, ones_e, (((1,), (0,)), ((), ())),
                               preferred_element_type=jnp.float32)
        return d + d_lo

    p = jnp.exp(l)
    zb = rowsum(p)
    probs = p / zb
    masked = probs * routing
    denom = rowsum(masked) + jnp.float32(1e-6)
    outp = masked / denom
    probs_ref[...] = outp
    imp_ref[...] = imp_ref[...] + jnp.sum(outp, axis=0, keepdims=True)
    load_ref[...] = load_ref[...] + jnp.sum(routing, axis=0, keepdims=True)

    @pl.when(i == nb - 1)
    def _():
        def cv2(x):
            mean = jnp.sum(x) / E
            var = jnp.sum((x - mean) ** 2) / (E - 1)
            return var / (mean + jnp.float32(1e-6)) ** 2

        aux_ref[...] = jnp.reshape(cv2(imp_ref[...]) + cv2(load_ref[...]),
                                   (1, 1))


def _router(flat, w, C, blk_mm=4096, blk_fin=1024):
    n, h = flat.shape
    e = w.shape[1]
    assert n % blk_mm == 0 and n % blk_fin == 0

    logits, logits_t = pl.pallas_call(
        _mm_body_t,
        grid=(n // blk_mm,),
        in_specs=[pl.BlockSpec((blk_mm, h), lambda i: (i, 0)),
                  pl.BlockSpec((h, e), lambda i: (0, 0))],
        out_specs=[pl.BlockSpec((blk_mm, e), lambda i: (i, 0)),
                   pl.BlockSpec((e, blk_mm), lambda i: (0, i))],
        out_shape=[jax.ShapeDtypeStruct((n, e), jnp.float32),
                   jax.ShapeDtypeStruct((e, n), jnp.float32)],
        compiler_params=pltpu.CompilerParams(
            dimension_semantics=("parallel",)),
    )(flat, w)
    sck, nw, epw = _sc_thresh(e, n, C)
    res = sck(logits_t)
    pairs = res[:, :2 * epw].reshape(e, 2)
    thr = pairs[:, 0].reshape(1, e)
    keq = pairs[:, 1].reshape(1, e)

    nb = n // blk_fin
    probs, routing, aux = pl.pallas_call(
        functools.partial(_final_body, E=e, nb=nb),
        grid=(nb,),
        in_specs=[pl.BlockSpec((blk_fin, e), lambda i: (i, 0)),
                  pl.BlockSpec((1, e), lambda i: (0, 0)),
                  pl.BlockSpec((1, e), lambda i: (0, 0))],
        out_specs=[pl.BlockSpec((blk_fin, e), lambda i: (i, 0)),
                   pl.BlockSpec((blk_fin, e), lambda i: (i, 0)),
                   pl.BlockSpec((1, 1), lambda i: (0, 0))],
        out_shape=[jax.ShapeDtypeStruct((n, e), jnp.float32),
                   jax.ShapeDtypeStruct((n, e), jnp.float32),
                   jax.ShapeDtypeStruct((1, 1), jnp.float32)],
        scratch_shapes=[pltpu.VMEM((1, e), jnp.int32),
                        pltpu.VMEM((1, e), jnp.float32),
                        pltpu.VMEM((1, e), jnp.float32)],
        compiler_params=pltpu.CompilerParams(
            dimension_semantics=("arbitrary",)),
    )(logits, thr, keq)

    return probs, routing, aux


def kernel(hidden_states, W):
    b, s, h = hidden_states.shape
    e = W.shape[1]
    flat = hidden_states.reshape(b * s, h)
    probs, routing, aux = _router(flat, W, _TOP_C)
    return (probs.reshape(b, s, e), routing.reshape(b, s, e), aux[0, 0])
